# baseline (speedup 1.0000x reference)
import functools

import jax
import jax.numpy as jnp
from jax import lax
from jax.experimental import pallas as pl
from jax.experimental.pallas import tpu as pltpu
from jax.experimental.pallas import tpu_sc as plsc

N = 10000
NP = 10240
PN = NP // 4
E = 320000
DIM = 32

NC = 2
NS = 16
NW = NC * NS

BATCH = 125
OPS_PER_TILE = (E // NW) // BATCH
ROWS_PER_TILE = NP // NS
NBUF = 4


@functools.cache
def _make_segsum_sc():
    mesh = plsc.VectorSubcoreMesh(core_axis_name="c", subcore_axis_name="s",
                                  num_cores=NC, num_subcores=NS)
    return pl.kernel(
        _segsum_sc_body,
        out_type=jax.ShapeDtypeStruct((NC, NP, DIM), jnp.float32),
        mesh=mesh,
        scratch_types=[
            pltpu.VMEM((OPS_PER_TILE, BATCH), jnp.int32),
            pltpu.VMEM((OPS_PER_TILE, BATCH), jnp.int32),
            [pltpu.VMEM((BATCH, DIM), jnp.float32)] * NBUF,
            [pltpu.SemaphoreType.DMA] * NBUF,
            pltpu.VMEM_SHARED((NP, DIM), jnp.float32),
        ],
        compiler_params=pltpu.CompilerParams(use_tc_tiling_on_sc=False),
    )


def _segsum_sc(p, edge3d, init, zeros):
    return _make_segsum_sc()(p, edge3d, init, zeros)


def _segsum_sc_body(p_hbm, edge_hbm, init_hbm, zeros_hbm, out_hbm,
                    src_idx, dst_idx, bufs, gsems, agg):
    c = lax.axis_index("c")
    s = lax.axis_index("s")

    row0 = s * ROWS_PER_TILE

    @pl.when(c == 0)
    def _init_self():
        pltpu.sync_copy(init_hbm.at[pl.ds(row0, ROWS_PER_TILE)],
                        agg.at[pl.ds(row0, ROWS_PER_TILE)])

    @pl.when(c != 0)
    def _init_zero():
        pltpu.sync_copy(zeros_hbm.at[pl.ds(row0, ROWS_PER_TILE)],
                        agg.at[pl.ds(row0, ROWS_PER_TILE)])

    idx_row0 = (c * NS + s) * OPS_PER_TILE
    pltpu.sync_copy(edge_hbm.at[0, pl.ds(idx_row0, OPS_PER_TILE)], src_idx)
    pltpu.sync_copy(edge_hbm.at[1, pl.ds(idx_row0, OPS_PER_TILE)], dst_idx)

    plsc.subcore_barrier()

    for b in range(NBUF):
        pltpu.async_copy(p_hbm.at[src_idx.at[b]], bufs[b], gsems[b])

    @pl.loop(0, OPS_PER_TILE // NBUF)
    def _edge_group(jj):
        for b in range(NBUF):
            j = jj * NBUF + b
            pltpu.make_async_copy(p_hbm.at[src_idx.at[j]], bufs[b],
                                  gsems[b]).wait()
            pltpu.sync_copy(bufs[b], agg.at[dst_idx.at[j]], add=True)

            @pl.when(jj < OPS_PER_TILE // NBUF - 1)
            def _next():
                pltpu.async_copy(p_hbm.at[src_idx.at[j + NBUF]], bufs[b],
                                 gsems[b])

    plsc.subcore_barrier()

    pltpu.sync_copy(agg.at[pl.ds(row0, ROWS_PER_TILE)],
                    out_hbm.at[c, pl.ds(row0, ROWS_PER_TILE)])


BN = 2000


def _mm0_body(x4_ref, wn_ref, ws_ref, bn_ref, wfc0_ref, p_ref, i_ref, t_ref):
    ps, is_, ts = [], [], []
    for j in range(4):
        xj = x4_ref[j]
        ps.append(jnp.dot(xj, wn_ref[...], preferred_element_type=jnp.float32))
        is_.append(bn_ref[...] + jnp.dot(xj, ws_ref[...],
                                         preferred_element_type=jnp.float32))
        ts.append(jnp.dot(xj, wfc0_ref[...],
                          preferred_element_type=jnp.float32))
    p_ref[...] = jnp.concatenate(ps, axis=1)
    i_ref[...] = jnp.concatenate(is_, axis=1)
    t_ref[...] = jnp.concatenate(ts, axis=1)


def _layer0_matmul(x4, Wn0, Ws0, bn0, Wfc0):
    n_classes = Wfc0.shape[1]
    return pl.pallas_call(
        _mm0_body,
        grid=(PN // BP,),
        in_specs=[
            pl.BlockSpec((4, BP, 128), lambda i: (0, i, 0)),
            pl.BlockSpec((128, DIM), lambda i: (0, 0)),
            pl.BlockSpec((128, DIM), lambda i: (0, 0)),
            pl.BlockSpec((1, DIM), lambda i: (0, 0)),
            pl.BlockSpec((128, n_classes), lambda i: (0, 0)),
        ],
        out_specs=[
            pl.BlockSpec((BP, 128), lambda i: (i, 0)),
            pl.BlockSpec((BP, 128), lambda i: (i, 0)),
            pl.BlockSpec((BP, 4 * n_classes), lambda i: (i, 0)),
        ],
        out_shape=[
            jax.ShapeDtypeStruct((PN, 128), jnp.float32),
            jax.ShapeDtypeStruct((PN, 128), jnp.float32),
            jax.ShapeDtypeStruct((PN, 4 * n_classes), jnp.float32),
        ],
    )(x4, Wn0, Ws0, bn0.reshape(1, DIM), Wfc0)


BP = 512


def _combine_body(agg_ref, wn_ref, ws_ref, bn_ref, h_ref, p_ref, i_ref):
    h = jnp.maximum(agg_ref[0] + agg_ref[1], 0.0)
    h_ref[...] = h
    p_ref[...] = jnp.dot(h, wn_ref[...], preferred_element_type=jnp.float32,
                         precision=lax.Precision.HIGHEST)
    i_ref[...] = bn_ref[...] + jnp.dot(h, ws_ref[...],
                                       preferred_element_type=jnp.float32,
                                       precision=lax.Precision.HIGHEST)


def _combine_matmul(agg_pk, Wn4, Ws4, bn4):
    return pl.pallas_call(
        _combine_body,
        grid=(PN // BP,),
        in_specs=[
            pl.BlockSpec((NC, BP, 128), lambda i: (0, i, 0)),
            pl.BlockSpec((128, 128), lambda i: (0, 0)),
            pl.BlockSpec((128, 128), lambda i: (0, 0)),
            pl.BlockSpec((1, 128), lambda i: (0, 0)),
        ],
        out_specs=[
            pl.BlockSpec((BP, 128), lambda i: (i, 0)),
            pl.BlockSpec((BP, 128), lambda i: (i, 0)),
            pl.BlockSpec((BP, 128), lambda i: (i, 0)),
        ],
        out_shape=[
            jax.ShapeDtypeStruct((PN, 128), jnp.float32),
            jax.ShapeDtypeStruct((PN, 128), jnp.float32),
            jax.ShapeDtypeStruct((PN, 128), jnp.float32),
        ],
    )(agg_pk, Wn4, Ws4, bn4)


def _final_body(tx_ref, h1_ref, h2_ref, agg_ref, w1_ref, w2_ref,
                w3_ref, bfc_ref, out_ref):
    h3 = jnp.maximum(agg_ref[0] + agg_ref[1], 0.0)
    hp = lax.Precision.HIGHEST
    t_pk = tx_ref[...] + bfc_ref[...]
    t_pk += jnp.dot(h1_ref[...], w1_ref[...],
                    preferred_element_type=jnp.float32, precision=hp)
    t_pk += jnp.dot(h2_ref[...], w2_ref[...],
                    preferred_element_type=jnp.float32, precision=hp)
    t_pk += jnp.dot(h3, w3_ref[...],
                    preferred_element_type=jnp.float32, precision=hp)
    n_classes = out_ref.shape[2]
    for j in range(4):
        t = t_pk[:, j * n_classes:(j + 1) * n_classes]
        m = jnp.max(t, axis=1, keepdims=True)
        lse = m + jnp.log(jnp.sum(jnp.exp(t - m), axis=1, keepdims=True))
        out_ref[j] = t - lse


def _final(tx_pk, h1_pk, h2_pk, agg2_pk, Wfc, bfc):
    n_classes = Wfc.shape[1]
    eye4 = jnp.eye(4, dtype=jnp.float32)
    out = pl.pallas_call(
        _final_body,
        grid=(PN // BP,),
        in_specs=[
            pl.BlockSpec((BP, 4 * n_classes), lambda i: (i, 0)),
            pl.BlockSpec((BP, 128), lambda i: (i, 0)),
            pl.BlockSpec((BP, 128), lambda i: (i, 0)),
            pl.BlockSpec((NC, BP, 128), lambda i: (0, i, 0)),
            pl.BlockSpec((128, 4 * n_classes), lambda i: (0, 0)),
            pl.BlockSpec((128, 4 * n_classes), lambda i: (0, 0)),
            pl.BlockSpec((128, 4 * n_classes), lambda i: (0, 0)),
            pl.BlockSpec((1, 4 * n_classes), lambda i: (0, 0)),
        ],
        out_specs=pl.BlockSpec((4, BP, n_classes), lambda i: (0, i, 0)),
        out_shape=jax.ShapeDtypeStruct((4, PN, n_classes), jnp.float32),
    )(tx_pk, h1_pk, h2_pk, agg2_pk,
      jnp.kron(eye4, Wfc[128:160]), jnp.kron(eye4, Wfc[160:192]),
      jnp.kron(eye4, Wfc[192:224]), jnp.tile(bfc, 4)[None])
    return out.reshape(NP, n_classes)[0:N]


def kernel(x, edge_index, Wn0, bn0, Ws0, Wn1, bn1, Ws1, Wn2, bn2, Ws2,
           Wfc, bfc):
    ev = 4 * (edge_index % PN) + edge_index // PN
    edge3d = ev.reshape(2, E // BATCH, BATCH)
    zeros = jnp.zeros((NP, DIM), jnp.float32)
    eye4 = jnp.eye(4, dtype=jnp.float32)

    def pk(v):
        return v.reshape(NP, DIM)

    x4 = jnp.pad(x, ((0, NP - N), (0, 0))).reshape(4, PN, 128)

    p0, init0, tx = _layer0_matmul(x4, Wn0, Ws0, bn0, Wfc[0:128])
    a0 = _segsum_sc(pk(p0), edge3d, pk(init0), zeros)

    h1, p1, i1 = _combine_matmul(a0.reshape(NC, PN, 128),
                                 jnp.kron(eye4, Wn1), jnp.kron(eye4, Ws1),
                                 jnp.tile(bn1, 4)[None])
    a1 = _segsum_sc(pk(p1), edge3d, pk(i1), zeros)

    h2, p2, i2 = _combine_matmul(a1.reshape(NC, PN, 128),
                                 jnp.kron(eye4, Wn2), jnp.kron(eye4, Ws2),
                                 jnp.tile(bn2, 4)[None])
    a2 = _segsum_sc(pk(p2), edge3d, pk(i2), zeros)

    return _final(tx, h1, h2, a2.reshape(NC, PN, 128), Wfc, bfc)


# speedup vs baseline: 1.0356x; 1.0356x over previous
import functools

import jax
import jax.numpy as jnp
from jax import lax
from jax.experimental import pallas as pl
from jax.experimental.pallas import tpu as pltpu
from jax.experimental.pallas import tpu_sc as plsc

N = 10000
NP = 10240
PN = NP // 4
E = 320000
DIM = 32

NC = 2
NS = 16
NW = NC * NS

BATCH = 125
OPS_PER_TILE = (E // NW) // BATCH
ROWS_PER_TILE = NP // NS
NBUF = 4


@functools.cache
def _make_segsum_sc():
    mesh = plsc.VectorSubcoreMesh(core_axis_name="c", subcore_axis_name="s",
                                  num_cores=NC, num_subcores=NS)
    return pl.kernel(
        _segsum_sc_body,
        out_type=jax.ShapeDtypeStruct((NC, NP, DIM), jnp.float32),
        mesh=mesh,
        scratch_types=[
            pltpu.VMEM((OPS_PER_TILE, BATCH), jnp.int32),
            pltpu.VMEM((OPS_PER_TILE, BATCH), jnp.int32),
            [pltpu.VMEM((BATCH, DIM), jnp.float32)] * NBUF,
            [pltpu.SemaphoreType.DMA] * NBUF,
            pltpu.VMEM_SHARED((NP, DIM), jnp.float32),
        ],
        compiler_params=pltpu.CompilerParams(use_tc_tiling_on_sc=False),
    )


def _segsum_sc(p, edge3d, init, zeros):
    return _make_segsum_sc()(p, edge3d, init, zeros)


def _segsum_sc_body(p_hbm, edge_hbm, init_hbm, zeros_hbm, out_hbm,
                    src_idx, dst_idx, bufs, gsems, agg):
    c = lax.axis_index("c")
    s = lax.axis_index("s")

    row0 = s * ROWS_PER_TILE

    @pl.when(c == 0)
    def _init_self():
        pltpu.sync_copy(init_hbm.at[pl.ds(row0, ROWS_PER_TILE)],
                        agg.at[pl.ds(row0, ROWS_PER_TILE)])

    @pl.when(c != 0)
    def _init_zero():
        pltpu.sync_copy(zeros_hbm.at[pl.ds(row0, ROWS_PER_TILE)],
                        agg.at[pl.ds(row0, ROWS_PER_TILE)])

    idx_row0 = (c * NS + s) * OPS_PER_TILE
    pltpu.sync_copy(edge_hbm.at[0, pl.ds(idx_row0, OPS_PER_TILE)], src_idx)
    pltpu.sync_copy(edge_hbm.at[1, pl.ds(idx_row0, OPS_PER_TILE)], dst_idx)

    plsc.subcore_barrier()

    for b in range(NBUF):
        pltpu.async_copy(p_hbm.at[src_idx.at[b]], bufs[b], gsems[b])

    @pl.loop(0, OPS_PER_TILE // NBUF)
    def _edge_group(jj):
        for b in range(NBUF):
            j = jj * NBUF + b
            pltpu.make_async_copy(p_hbm.at[src_idx.at[j]], bufs[b],
                                  gsems[b]).wait()
            pltpu.sync_copy(bufs[b], agg.at[dst_idx.at[j]], add=True)

            @pl.when(jj < OPS_PER_TILE // NBUF - 1)
            def _next():
                pltpu.async_copy(p_hbm.at[src_idx.at[j + NBUF]], bufs[b],
                                 gsems[b])

    plsc.subcore_barrier()

    pltpu.sync_copy(agg.at[pl.ds(row0, ROWS_PER_TILE)],
                    out_hbm.at[c, pl.ds(row0, ROWS_PER_TILE)])


BN = 2000


def _mm0_body(x4_ref, wn_ref, ws_ref, bn_ref, wfc0_ref, p_ref, i_ref, t_ref):
    ps, is_, ts = [], [], []
    for j in range(4):
        xj = x4_ref[j]
        ps.append(jnp.dot(xj, wn_ref[...], preferred_element_type=jnp.float32))
        is_.append(bn_ref[...] + jnp.dot(xj, ws_ref[...],
                                         preferred_element_type=jnp.float32))
        ts.append(jnp.dot(xj, wfc0_ref[...],
                          preferred_element_type=jnp.float32))
    p_ref[...] = jnp.concatenate(ps, axis=1)
    i_ref[...] = jnp.concatenate(is_, axis=1)
    t_ref[...] = jnp.concatenate(ts, axis=1)


def _layer0_matmul(x4, Wn0, Ws0, bn0, Wfc0):
    n_classes = Wfc0.shape[1]
    return pl.pallas_call(
        _mm0_body,
        grid=(PN // BP,),
        in_specs=[
            pl.BlockSpec((4, BP, 128), lambda i: (0, i, 0)),
            pl.BlockSpec((128, DIM), lambda i: (0, 0)),
            pl.BlockSpec((128, DIM), lambda i: (0, 0)),
            pl.BlockSpec((1, DIM), lambda i: (0, 0)),
            pl.BlockSpec((128, n_classes), lambda i: (0, 0)),
        ],
        out_specs=[
            pl.BlockSpec((BP, 128), lambda i: (i, 0)),
            pl.BlockSpec((BP, 128), lambda i: (i, 0)),
            pl.BlockSpec((BP, 4 * n_classes), lambda i: (i, 0)),
        ],
        out_shape=[
            jax.ShapeDtypeStruct((PN, 128), jnp.float32),
            jax.ShapeDtypeStruct((PN, 128), jnp.float32),
            jax.ShapeDtypeStruct((PN, 4 * n_classes), jnp.float32),
        ],
    )(x4, Wn0, Ws0, bn0.reshape(1, DIM), Wfc0)


BP = 512


def _combine_body(agg_ref, wn_ref, ws_ref, bn_ref, h_ref, p_ref, i_ref):
    h = jnp.maximum(agg_ref[0] + agg_ref[1], 0.0)
    h_ref[...] = h
    p_ref[...] = jnp.dot(h, wn_ref[...], preferred_element_type=jnp.float32,
                         precision=lax.Precision.HIGHEST)
    i_ref[...] = bn_ref[...] + jnp.dot(h, ws_ref[...],
                                       preferred_element_type=jnp.float32,
                                       precision=lax.Precision.HIGHEST)


def _combine_matmul(agg_pk, Wn4, Ws4, bn4):
    return pl.pallas_call(
        _combine_body,
        grid=(PN // BP,),
        in_specs=[
            pl.BlockSpec((NC, BP, 128), lambda i: (0, i, 0)),
            pl.BlockSpec((128, 128), lambda i: (0, 0)),
            pl.BlockSpec((128, 128), lambda i: (0, 0)),
            pl.BlockSpec((1, 128), lambda i: (0, 0)),
        ],
        out_specs=[
            pl.BlockSpec((BP, 128), lambda i: (i, 0)),
            pl.BlockSpec((BP, 128), lambda i: (i, 0)),
            pl.BlockSpec((BP, 128), lambda i: (i, 0)),
        ],
        out_shape=[
            jax.ShapeDtypeStruct((PN, 128), jnp.float32),
            jax.ShapeDtypeStruct((PN, 128), jnp.float32),
            jax.ShapeDtypeStruct((PN, 128), jnp.float32),
        ],
    )(agg_pk, Wn4, Ws4, bn4)


def _fcpart_body(acc_ref, h_ref, w_ref, out_ref):
    out_ref[...] = acc_ref[...] + jnp.dot(
        h_ref[...], w_ref[...], preferred_element_type=jnp.float32,
        precision=lax.Precision.HIGHEST)


def _fc_partial(acc, h_pk, W4):
    w = acc.shape[1]
    return pl.pallas_call(
        _fcpart_body,
        grid=(PN // BP,),
        in_specs=[
            pl.BlockSpec((BP, w), lambda i: (i, 0)),
            pl.BlockSpec((BP, 128), lambda i: (i, 0)),
            pl.BlockSpec((128, w), lambda i: (0, 0)),
        ],
        out_specs=pl.BlockSpec((BP, w), lambda i: (i, 0)),
        out_shape=jax.ShapeDtypeStruct((PN, w), jnp.float32),
    )(acc, h_pk, W4)


def _final_body(tp_ref, agg_ref, w3_ref, bfc_ref, out_ref):
    h3 = jnp.maximum(agg_ref[0] + agg_ref[1], 0.0)
    hp = lax.Precision.HIGHEST
    t_pk = tp_ref[...] + bfc_ref[...]
    t_pk += jnp.dot(h3, w3_ref[...],
                    preferred_element_type=jnp.float32, precision=hp)
    n_classes = out_ref.shape[2]
    for j in range(4):
        t = t_pk[:, j * n_classes:(j + 1) * n_classes]
        m = jnp.max(t, axis=1, keepdims=True)
        lse = m + jnp.log(jnp.sum(jnp.exp(t - m), axis=1, keepdims=True))
        out_ref[j] = t - lse


def _final(tp_pk, agg2_pk, Wfc, bfc):
    n_classes = Wfc.shape[1]
    eye4 = jnp.eye(4, dtype=jnp.float32)
    out = pl.pallas_call(
        _final_body,
        grid=(PN // BP,),
        in_specs=[
            pl.BlockSpec((BP, 4 * n_classes), lambda i: (i, 0)),
            pl.BlockSpec((NC, BP, 128), lambda i: (0, i, 0)),
            pl.BlockSpec((128, 4 * n_classes), lambda i: (0, 0)),
            pl.BlockSpec((1, 4 * n_classes), lambda i: (0, 0)),
        ],
        out_specs=pl.BlockSpec((4, BP, n_classes), lambda i: (0, i, 0)),
        out_shape=jax.ShapeDtypeStruct((4, PN, n_classes), jnp.float32),
    )(tp_pk, agg2_pk, jnp.kron(eye4, Wfc[192:224]), jnp.tile(bfc, 4)[None])
    return out.reshape(NP, n_classes)[0:N]


def kernel(x, edge_index, Wn0, bn0, Ws0, Wn1, bn1, Ws1, Wn2, bn2, Ws2,
           Wfc, bfc):
    ev = 4 * (edge_index % PN) + edge_index // PN
    edge3d = ev.reshape(2, E // BATCH, BATCH)
    zeros = jnp.zeros((NP, DIM), jnp.float32)
    eye4 = jnp.eye(4, dtype=jnp.float32)

    def pk(v):
        return v.reshape(NP, DIM)

    x4 = jnp.pad(x, ((0, NP - N), (0, 0))).reshape(4, PN, 128)

    p0, init0, tx = _layer0_matmul(x4, Wn0, Ws0, bn0, Wfc[0:128])
    a0 = _segsum_sc(pk(p0), edge3d, pk(init0), zeros)

    h1, p1, i1 = _combine_matmul(a0.reshape(NC, PN, 128),
                                 jnp.kron(eye4, Wn1), jnp.kron(eye4, Ws1),
                                 jnp.tile(bn1, 4)[None])
    a1 = _segsum_sc(pk(p1), edge3d, pk(i1), zeros)
    tp1 = _fc_partial(tx, h1, jnp.kron(eye4, Wfc[128:160]))

    h2, p2, i2 = _combine_matmul(a1.reshape(NC, PN, 128),
                                 jnp.kron(eye4, Wn2), jnp.kron(eye4, Ws2),
                                 jnp.tile(bn2, 4)[None])
    a2 = _segsum_sc(pk(p2), edge3d, pk(i2), zeros)
    tp2 = _fc_partial(tp1, h2, jnp.kron(eye4, Wfc[160:192]))

    return _final(tp2, a2.reshape(NC, PN, 128), Wfc, bfc)
